# baseline (speedup 1.0000x reference)
import functools

import jax
import jax.numpy as jnp
from jax import lax
from jax.experimental import pallas as pl
from jax.experimental.pallas import tpu as pltpu
from jax.experimental.pallas import tpu_sc as plsc

B = 16384
L = 200
D = 64
NUM_CORES = 2
NUM_SUBCORES = 16
NUM_WORKERS = NUM_CORES * NUM_SUBCORES
SEQ_PER_W = B // NUM_WORKERS
SEQ_CB = 2
CHUNKS_PER_W = SEQ_PER_W // SEQ_CB
SPS = 64
CPS = SPS // SEQ_CB
NUM_SUPERS = SEQ_PER_W // SPS

_mesh = plsc.VectorSubcoreMesh(core_axis_name="c", subcore_axis_name="s")


@functools.partial(
    pl.kernel,
    mesh=_mesh,
    out_type=jax.ShapeDtypeStruct((B, L, D), jnp.float32),
    scratch_types=[
        pltpu.VMEM((SPS, L), jnp.int32),
        pltpu.VMEM((SPS, L), jnp.int32),
        pltpu.VMEM((SEQ_CB, L, D), jnp.float32),
        pltpu.VMEM((SEQ_CB, L, D), jnp.float32),
        pltpu.SemaphoreType.DMA,
        pltpu.SemaphoreType.DMA,
        pltpu.SemaphoreType.DMA,
        pltpu.SemaphoreType.DMA,
    ],
    compiler_params=pltpu.CompilerParams(use_tc_tiling_on_sc=False),
)
def _emb_lookup(idx_hbm, table_hbm, out_hbm,
                idx_v0, idx_v1, rows0, rows1, sg0, sg1, ss0, ss1):
    wid = lax.axis_index("s") * NUM_CORES + lax.axis_index("c")
    wseq = wid * SEQ_PER_W
    idxb = (idx_v0, idx_v1)
    rows = (rows0, rows1)
    sg = (sg0, sg1)
    ss = (ss0, ss1)

    def gather_start(sb, local_chunk, b):
        for jj in range(SEQ_CB):
            pltpu.async_copy(
                table_hbm.at[idxb[sb].at[local_chunk * SEQ_CB + jj]],
                rows[b].at[jj], sg[b])

    def gather_wait(b):
        for jj in range(SEQ_CB):
            pltpu.make_async_copy(
                table_hbm.at[idx_v0.at[0]], rows[b].at[jj], sg[b]).wait()

    def store_start(seq0, b):
        pltpu.async_copy(rows[b], out_hbm.at[pl.ds(seq0, SEQ_CB)], ss[b])

    def store_wait(b):
        pltpu.make_async_copy(
            rows[b], out_hbm.at[pl.ds(wseq, SEQ_CB)], ss[b]).wait()

    pltpu.sync_copy(idx_hbm.at[pl.ds(wseq, SPS)], idx_v0)
    gather_start(0, 0, 1)
    pltpu.async_copy(rows0, out_hbm.at[pl.ds(wseq, SEQ_CB)], ss0)

    def super_pair(sp, _):
        for sb in (0, 1):
            s = 2 * sp + sb
            pltpu.sync_copy(idx_hbm.at[pl.ds(wseq + s * SPS, SPS)], idxb[sb])

            def chunk_pair(p, _):
                for b in (0, 1):
                    lc = 2 * p + b
                    c = s * CPS + lc
                    store_wait(b)
                    gather_start(sb, lc, b)
                    gather_wait(1 - b)
                    prev = wseq + jnp.maximum(c - 1, 0) * SEQ_CB
                    store_start(prev, 1 - b)
                return 0

            lax.fori_loop(0, CPS // 2, chunk_pair, 0)
        return 0

    lax.fori_loop(0, NUM_SUPERS // 2, super_pair, 0)

    gather_wait(1)
    store_start(wseq + (CHUNKS_PER_W - 1) * SEQ_CB, 1)
    store_wait(0)
    store_wait(1)


def kernel(indices, table):
    return _emb_lookup(indices, table)
